# baseline (speedup 1.0000x reference)
import functools

import jax
import jax.numpy as jnp
from jax import lax
from jax.experimental import pallas as pl
from jax.experimental.pallas import tpu as pltpu
from jax.experimental.pallas import tpu_sc as plsc

NC = 2
NS = 16

def _deg_body(dst_hbm, out_hbm, hist, dstw):
    c = lax.axis_index("c")
    s = lax.axis_index("s")
    w = c * NS + s
    npad = hist.shape[0]
    nch = dstw.shape[0] // 16
    zero16 = jnp.zeros((16,), jnp.float32)

    def zstep(i, carry):
        hist[pl.ds(i * 16, 16)] = zero16
        return carry

    lax.fori_loop(0, npad // 16, zstep, 0)
    pltpu.sync_copy(dst_hbm.at[w], dstw)
    ones = jnp.ones((16,), jnp.float32)

    def step(i, carry):
        idx = dstw[pl.ds(i * 16, 16)]
        plsc.addupdate_scatter(hist, [idx], ones)
        return carry

    lax.fori_loop(0, nch, step, 0)
    pltpu.sync_copy(hist, out_hbm.at[w])


def _degree_partials(dst_tiled, npad):
    mesh = plsc.VectorSubcoreMesh(
        core_axis_name="c", subcore_axis_name="s", num_cores=NC, num_subcores=NS)
    epw = dst_tiled.shape[1]
    fn = pl.kernel(
        _deg_body,
        out_type=jax.ShapeDtypeStruct((NC * NS, npad), jnp.float32),
        mesh=mesh,
        compiler_params=pltpu.CompilerParams(needs_layout_passes=False),
        scratch_types=[
            pltpu.VMEM((npad,), jnp.float32),
            pltpu.VMEM((epw,), jnp.int32),
        ],
    )
    return fn(dst_tiled)


def _mm_body(deg_ref, x_ref, w_ref, z2_ref, inv_ref):
    d = jnp.sum(deg_ref[...], axis=0) + 1.0
    inv = lax.rsqrt(d)
    xn = x_ref[...] * inv[:, None]
    z = jnp.dot(xn, w_ref[...], preferred_element_type=jnp.float32)
    h = z.shape[1] // 2
    z2_ref[0] = z[:, :h]
    z2_ref[1] = z[:, h:]
    inv_ref[...] = inv[:, None]


def _matmul(degp, x, W, npad, bn=1024):
    n, d = x.shape
    u = W.shape[1]
    h = u // 2
    grid = (-(-npad // bn),)
    return pl.pallas_call(
        _mm_body,
        grid=grid,
        in_specs=[
            pl.BlockSpec((NC * NS, bn), lambda i: (0, i)),
            pl.BlockSpec((bn, d), lambda i: (i, 0)),
            pl.BlockSpec((d, u), lambda i: (0, 0)),
        ],
        out_specs=[
            pl.BlockSpec((NC, bn, h), lambda i: (0, i, 0)),
            pl.BlockSpec((bn, 1), lambda i: (i, 0)),
        ],
        out_shape=[
            jax.ShapeDtypeStruct((NC, npad, h), jnp.float32),
            jax.ShapeDtypeStruct((n, 1), jnp.float32),
        ],
    )(degp, x, W)


def _scat_body(srcoff_hbm, dst_hbm, zflat_hbm, out_hbm,
               acc, srcb, dstb, r0, r1, sg, s0, s1, si0, si1, si2, si3):
    c = lax.axis_index("c")
    s = lax.axis_index("s")
    npad = acc.shape[0]
    npt = npad // NS
    nchunk = srcb.shape[0]
    pltpu.sync_copy(zflat_hbm.at[pl.ds(c * npad + s * npt, npt)],
                    acc.at[pl.ds(s * npt, npt)])
    pltpu.sync_copy(srcoff_hbm.at[c, s], srcb)
    plsc.subcore_barrier()

    def iload(j, k, sem):
        pltpu.async_copy(dst_hbm.at[s, j], dstb.at[k], sem)

    def iwait(k, sem):
        pltpu.make_async_copy(dst_hbm.at[s, 0], dstb.at[k], sem).wait()

    def gather(j, rows):
        pltpu.async_copy(zflat_hbm.at[srcb.at[j]], rows, sg).wait()

    def ascatter(k, rows, sem):
        pltpu.async_copy(rows, acc.at[dstb.at[k]], sem, add=True)

    def swait(rows, sem):
        pltpu.make_async_copy(rows, acc.at[dstb.at[0]], sem).wait()

    iload(0, 0, si0)
    iload(1, 1, si1)
    iload(2, 2, si2)
    iload(3, 3, si3)
    iwait(0, si0)
    gather(0, r0)
    ascatter(0, r0, s0)
    iwait(1, si1)
    gather(1, r1)
    ascatter(1, r1, s1)
    swait(r0, s0)
    iwait(2, si2)
    gather(2, r0)
    ascatter(2, r0, s0)
    iload(4, 0, si0)
    swait(r1, s1)
    iwait(3, si3)
    gather(3, r1)
    ascatter(3, r1, s1)
    iload(5, 1, si1)

    def body(jj, carry):
        j = jj * 4
        swait(r0, s0)
        iwait(0, si0)
        gather(j, r0)
        ascatter(0, r0, s0)
        iload(j + 2, 2, si2)
        swait(r1, s1)
        iwait(1, si1)
        gather(j + 1, r1)
        ascatter(1, r1, s1)
        iload(j + 3, 3, si3)
        swait(r0, s0)
        iwait(2, si2)
        gather(j + 2, r0)
        ascatter(2, r0, s0)

        @pl.when(j + 4 < nchunk)
        def _():
            iload(j + 4, 0, si0)

        swait(r1, s1)
        iwait(3, si3)
        gather(j + 3, r1)
        ascatter(3, r1, s1)

        @pl.when(j + 5 < nchunk)
        def _():
            iload(j + 5, 1, si1)

        return carry

    lax.fori_loop(1, nchunk // 4, body, 0)
    swait(r0, s0)
    swait(r1, s1)
    plsc.subcore_barrier()
    pltpu.sync_copy(acc.at[pl.ds(s * npt, npt)], out_hbm.at[c, pl.ds(s * npt, npt)])


def _message_pass(srcoff, dst_tiled, z2, npad, h):
    mesh = plsc.VectorSubcoreMesh(
        core_axis_name="c", subcore_axis_name="s", num_cores=NC, num_subcores=NS)
    nchunk, ch = dst_tiled.shape[1], dst_tiled.shape[2]
    zflat = z2.reshape(NC * npad, h)
    fn = pl.kernel(
        _scat_body,
        out_type=jax.ShapeDtypeStruct((NC, npad, h), jnp.float32),
        mesh=mesh,
        scratch_types=[
            pltpu.VMEM_SHARED((npad, h), jnp.float32),
            pltpu.VMEM((nchunk, ch), jnp.int32),
            pltpu.VMEM((4, ch), jnp.int32),
            pltpu.VMEM((ch, h), jnp.float32),
            pltpu.VMEM((ch, h), jnp.float32),
        ] + [pltpu.SemaphoreType.DMA] * 7,
    )
    return fn(srcoff, dst_tiled, zflat)


def _ep_body(p_ref, inv_ref, b_ref, o_ref):
    c = pl.program_id(1)
    bb = jnp.where(c == 0, b_ref[0], b_ref[1])
    o_ref[...] = jnp.maximum(p_ref[0] * inv_ref[...] + bb, 0.0)


def _epilogue(pooled, inv, b2, n, u, bn=2000):
    h = u // 2
    grid = (n // bn, NC)
    return pl.pallas_call(
        _ep_body,
        grid=grid,
        in_specs=[
            pl.BlockSpec((1, bn, h), lambda i, c: (c, i, 0)),
            pl.BlockSpec((bn, 1), lambda i, c: (i, 0)),
            pl.BlockSpec((NC, h), lambda i, c: (0, 0)),
        ],
        out_specs=pl.BlockSpec((bn, h), lambda i, c: (i, c)),
        out_shape=jax.ShapeDtypeStruct((n, u), jnp.float32),
    )(pooled, inv, b2)


def kernel(x, edge_index, W, b):
    n, d = x.shape
    u = W.shape[1]
    h = u // 2
    e = edge_index.shape[1]
    src = edge_index[0].astype(jnp.int32)
    dst = edge_index[1].astype(jnp.int32)
    npad = -(-n // (NS * 8)) * (NS * 8)

    nw = NC * NS
    epw = -(-e // (nw * 16)) * 16
    dst_deg = jnp.concatenate(
        [dst, jnp.full((nw * epw - e,), npad - 1, jnp.int32)]).reshape(nw, epw)

    ch = 128
    nch = -(-(e // NS) // (4 * ch)) * 4
    epad = NS * nch * ch - e
    srcp = jnp.concatenate([src, jnp.full((epad,), npad - 1, jnp.int32)])
    dstp = jnp.concatenate([dst, jnp.full((epad,), npad - 1, jnp.int32)])
    srcoff = jnp.stack([srcp, srcp + npad]).reshape(NC, NS, nch, ch)
    dst_sc = dstp.reshape(NS, nch, ch)

    degp = _degree_partials(dst_deg, npad)
    z2, inv = _matmul(degp, x, W, npad)
    pooled = _message_pass(srcoff, dst_sc, z2, npad, h)
    return _epilogue(pooled, inv, b.reshape(NC, h), n, u)


# speedup vs baseline: 1.9424x; 1.9424x over previous
import functools

import jax
import jax.numpy as jnp
from jax import lax
from jax.experimental import pallas as pl
from jax.experimental.pallas import tpu as pltpu
from jax.experimental.pallas import tpu_sc as plsc

NC = 2
NS = 16

def _deg_body(dst_hbm, out_hbm, hist, dstw):
    c = lax.axis_index("c")
    s = lax.axis_index("s")
    w = c * NS + s
    npad = hist.shape[0]
    nch = dstw.shape[0] // 16
    zero16 = jnp.zeros((16,), jnp.float32)

    def zstep(i, carry):
        hist[pl.ds(i * 16, 16)] = zero16
        return carry

    lax.fori_loop(0, npad // 16, zstep, 0)
    pltpu.sync_copy(dst_hbm.at[w], dstw)
    ones = jnp.ones((16,), jnp.float32)

    def step(i, carry):
        idx = dstw[pl.ds(i * 16, 16)]
        plsc.addupdate_scatter(hist, [idx], ones)
        return carry

    lax.fori_loop(0, nch, step, 0)
    pltpu.sync_copy(hist, out_hbm.at[w])


def _degree_partials(dst_tiled, npad):
    mesh = plsc.VectorSubcoreMesh(
        core_axis_name="c", subcore_axis_name="s", num_cores=NC, num_subcores=NS)
    epw = dst_tiled.shape[1]
    fn = pl.kernel(
        _deg_body,
        out_type=jax.ShapeDtypeStruct((NC * NS, npad), jnp.float32),
        mesh=mesh,
        compiler_params=pltpu.CompilerParams(needs_layout_passes=False),
        scratch_types=[
            pltpu.VMEM((npad,), jnp.float32),
            pltpu.VMEM((epw,), jnp.int32),
        ],
    )
    return fn(dst_tiled)


def _mm_body(deg_ref, x_ref, w_ref, z2_ref, inv_ref):
    d = jnp.sum(deg_ref[...], axis=0) + 1.0
    inv = lax.rsqrt(d)
    xn = x_ref[...] * inv[:, None]
    z = jnp.dot(xn, w_ref[...], preferred_element_type=jnp.float32)
    h = z.shape[1] // 2
    z2_ref[0] = z[:, :h]
    z2_ref[1] = z[:, h:]
    inv_ref[...] = inv[:, None]


def _matmul(degp, x, W, npad, bn=1024):
    n, d = x.shape
    u = W.shape[1]
    h = u // 2
    grid = (-(-npad // bn),)
    return pl.pallas_call(
        _mm_body,
        grid=grid,
        in_specs=[
            pl.BlockSpec((NC * NS, bn), lambda i: (0, i)),
            pl.BlockSpec((bn, d), lambda i: (i, 0)),
            pl.BlockSpec((d, u), lambda i: (0, 0)),
        ],
        out_specs=[
            pl.BlockSpec((NC, bn, h), lambda i: (0, i, 0)),
            pl.BlockSpec((bn, 1), lambda i: (i, 0)),
        ],
        out_shape=[
            jax.ShapeDtypeStruct((NC, npad, h), jnp.float32),
            jax.ShapeDtypeStruct((n, 1), jnp.float32),
        ],
    )(degp, x, W)


def _scat_body(srcoff_hbm, dst_hbm, zflat_hbm, out_hbm,
               acc, srcb, dstb, r0, r1, sg, s0, s1, si0, si1, si2, si3):
    c = lax.axis_index("c")
    s = lax.axis_index("s")
    npad = acc.shape[0]
    npt = npad // NS
    nchunk = srcb.shape[0]
    pltpu.sync_copy(zflat_hbm.at[pl.ds(c * npad + s * npt, npt)],
                    acc.at[pl.ds(s * npt, npt)])
    pltpu.sync_copy(srcoff_hbm.at[c, s], srcb)
    plsc.subcore_barrier()

    def iload(j, k, sem):
        pltpu.async_copy(dst_hbm.at[s, j], dstb.at[k], sem)

    def iwait(k, sem):
        pltpu.make_async_copy(dst_hbm.at[s, 0], dstb.at[k], sem).wait()

    def gather(j, rows):
        pltpu.async_copy(zflat_hbm.at[srcb.at[j]], rows, sg).wait()

    def ascatter(k, rows, sem):
        pltpu.async_copy(rows, acc.at[dstb.at[k]], sem, add=True)

    def swait(rows, sem):
        pltpu.make_async_copy(rows, acc.at[dstb.at[0]], sem).wait()

    iload(0, 0, si0)
    iload(1, 1, si1)
    iload(2, 2, si2)
    iload(3, 3, si3)
    iwait(0, si0)
    gather(0, r0)
    ascatter(0, r0, s0)
    iwait(1, si1)
    gather(1, r1)
    ascatter(1, r1, s1)
    swait(r0, s0)
    iwait(2, si2)
    gather(2, r0)
    ascatter(2, r0, s0)
    iload(4, 0, si0)
    swait(r1, s1)
    iwait(3, si3)
    gather(3, r1)
    ascatter(3, r1, s1)
    iload(5, 1, si1)

    def body(jj, carry):
        j = jj * 4
        swait(r0, s0)
        iwait(0, si0)
        gather(j, r0)
        ascatter(0, r0, s0)
        iload(j + 2, 2, si2)
        swait(r1, s1)
        iwait(1, si1)
        gather(j + 1, r1)
        ascatter(1, r1, s1)
        iload(j + 3, 3, si3)
        swait(r0, s0)
        iwait(2, si2)
        gather(j + 2, r0)
        ascatter(2, r0, s0)

        @pl.when(j + 4 < nchunk)
        def _():
            iload(j + 4, 0, si0)

        swait(r1, s1)
        iwait(3, si3)
        gather(j + 3, r1)
        ascatter(3, r1, s1)

        @pl.when(j + 5 < nchunk)
        def _():
            iload(j + 5, 1, si1)

        return carry

    lax.fori_loop(1, nchunk // 4, body, 0)
    swait(r0, s0)
    swait(r1, s1)
    plsc.subcore_barrier()
    pltpu.sync_copy(acc.at[pl.ds(s * npt, npt)], out_hbm.at[c, pl.ds(s * npt, npt)])


def _message_pass(srcoff, dst_tiled, z2, npad, h):
    mesh = plsc.VectorSubcoreMesh(
        core_axis_name="c", subcore_axis_name="s", num_cores=NC, num_subcores=NS)
    nchunk, ch = dst_tiled.shape[1], dst_tiled.shape[2]
    zflat = z2.reshape(NC * npad, h)
    fn = pl.kernel(
        _scat_body,
        out_type=jax.ShapeDtypeStruct((NC, npad, h), jnp.float32),
        mesh=mesh,
        scratch_types=[
            pltpu.VMEM_SHARED((npad, h), jnp.float32),
            pltpu.VMEM((nchunk, ch), jnp.int32),
            pltpu.VMEM((4, ch), jnp.int32),
            pltpu.VMEM((ch, h), jnp.float32),
            pltpu.VMEM((ch, h), jnp.float32),
        ] + [pltpu.SemaphoreType.DMA] * 7,
    )
    return fn(srcoff, dst_tiled, zflat)


def _ep_body(p_ref, inv_ref, b_ref, o_ref):
    c = pl.program_id(1)
    bb = jnp.where(c == 0, b_ref[0], b_ref[1])
    o_ref[...] = jnp.maximum(p_ref[0] * inv_ref[...] + bb, 0.0)


def _epilogue(pooled, inv, b2, n, u, bn=2000):
    h = u // 2
    grid = (n // bn, NC)
    return pl.pallas_call(
        _ep_body,
        grid=grid,
        in_specs=[
            pl.BlockSpec((1, bn, h), lambda i, c: (c, i, 0)),
            pl.BlockSpec((bn, 1), lambda i, c: (i, 0)),
            pl.BlockSpec((NC, h), lambda i, c: (0, 0)),
        ],
        out_specs=pl.BlockSpec((bn, h), lambda i, c: (i, c)),
        out_shape=jax.ShapeDtypeStruct((n, u), jnp.float32),
    )(pooled, inv, b2)


def kernel(x, edge_index, W, b):
    n, d = x.shape
    u = W.shape[1]
    h = u // 2
    e = edge_index.shape[1]
    src = edge_index[0].astype(jnp.int32)
    dst = edge_index[1].astype(jnp.int32)
    npad = -(-n // (NS * 8)) * (NS * 8)

    nw = NC * NS
    epw = -(-e // (nw * 16)) * 16
    dst_deg = jnp.concatenate(
        [dst, jnp.full((nw * epw - e,), npad - 1, jnp.int32)]).reshape(nw, epw)

    ch = 100
    nch = -(-(e // NS) // (4 * ch)) * 4
    epad = NS * nch * ch - e
    srcp = jnp.concatenate([src, jnp.full((epad,), npad - 1, jnp.int32)])
    dstp = jnp.concatenate([dst, jnp.full((epad,), npad - 1, jnp.int32)])
    srcoff = jnp.stack([srcp, srcp + npad]).reshape(NC, NS, nch, ch)
    dst_sc = dstp.reshape(NS, nch, ch)

    degp = _degree_partials(dst_deg, npad)
    z2, inv = _matmul(degp, x, W, npad)
    pooled = _message_pass(srcoff, dst_sc, z2, npad, h)
    return _epilogue(pooled, inv, b.reshape(NC, h), n, u)


# speedup vs baseline: 2.0033x; 1.0314x over previous
import functools

import jax
import jax.numpy as jnp
from jax import lax
from jax.experimental import pallas as pl
from jax.experimental.pallas import tpu as pltpu
from jax.experimental.pallas import tpu_sc as plsc

NC = 2
NS = 16

def _deg_body(dst_hbm, out_hbm, hist, dstw):
    c = lax.axis_index("c")
    s = lax.axis_index("s")
    w = c * NS + s
    npad = hist.shape[0]
    nch = dstw.shape[0] // 16
    zero16 = jnp.zeros((16,), jnp.float32)

    def zstep(i, carry):
        hist[pl.ds(i * 16, 16)] = zero16
        return carry

    lax.fori_loop(0, npad // 16, zstep, 0)
    pltpu.sync_copy(dst_hbm.at[w], dstw)
    ones = jnp.ones((16,), jnp.float32)

    def step(i, carry):
        idx = dstw[pl.ds(i * 16, 16)]
        plsc.addupdate_scatter(hist, [idx], ones)
        return carry

    lax.fori_loop(0, nch, step, 0)
    pltpu.sync_copy(hist, out_hbm.at[w])


def _degree_partials(dst_tiled, npad):
    mesh = plsc.VectorSubcoreMesh(
        core_axis_name="c", subcore_axis_name="s", num_cores=NC, num_subcores=NS)
    epw = dst_tiled.shape[1]
    fn = pl.kernel(
        _deg_body,
        out_type=jax.ShapeDtypeStruct((NC * NS, npad), jnp.float32),
        mesh=mesh,
        compiler_params=pltpu.CompilerParams(needs_layout_passes=False),
        scratch_types=[
            pltpu.VMEM((npad,), jnp.float32),
            pltpu.VMEM((epw,), jnp.int32),
        ],
    )
    return fn(dst_tiled)


def _mm_body(deg_ref, x_ref, w_ref, z2_ref, inv_ref):
    d = jnp.sum(deg_ref[...], axis=0) + 1.0
    inv = lax.rsqrt(d)
    xn = x_ref[...] * inv[:, None]
    z = jnp.dot(xn, w_ref[...], preferred_element_type=jnp.float32)
    h = z.shape[1] // 2
    z2_ref[0] = z[:, :h]
    z2_ref[1] = z[:, h:]
    inv_ref[...] = inv[:, None]


def _matmul(degp, x, W, npad, bn=2048):
    n, d = x.shape
    u = W.shape[1]
    h = u // 2
    grid = (-(-npad // bn),)
    return pl.pallas_call(
        _mm_body,
        grid=grid,
        in_specs=[
            pl.BlockSpec((NC * NS, bn), lambda i: (0, i)),
            pl.BlockSpec((bn, d), lambda i: (i, 0)),
            pl.BlockSpec((d, u), lambda i: (0, 0)),
        ],
        out_specs=[
            pl.BlockSpec((NC, bn, h), lambda i: (0, i, 0)),
            pl.BlockSpec((bn, 1), lambda i: (i, 0)),
        ],
        out_shape=[
            jax.ShapeDtypeStruct((NC, npad, h), jnp.float32),
            jax.ShapeDtypeStruct((n, 1), jnp.float32),
        ],
    )(degp, x, W)


def _scat_body(srcoff_hbm, dst_hbm, zflat_hbm, out_hbm,
               acc, srcb, dstb, r0, r1, sg, s0, s1, si0, si1, si2, si3):
    c = lax.axis_index("c")
    s = lax.axis_index("s")
    npad = acc.shape[0]
    npt = npad // NS
    nchunk = srcb.shape[0]
    pltpu.sync_copy(zflat_hbm.at[pl.ds(c * npad + s * npt, npt)],
                    acc.at[pl.ds(s * npt, npt)])
    pltpu.sync_copy(srcoff_hbm.at[c, s], srcb)
    plsc.subcore_barrier()

    def iload(j, k, sem):
        pltpu.async_copy(dst_hbm.at[s, j], dstb.at[k], sem)

    def iwait(k, sem):
        pltpu.make_async_copy(dst_hbm.at[s, 0], dstb.at[k], sem).wait()

    def gather(j, rows):
        pltpu.async_copy(zflat_hbm.at[srcb.at[j]], rows, sg).wait()

    def ascatter(k, rows, sem):
        pltpu.async_copy(rows, acc.at[dstb.at[k]], sem, add=True)

    def swait(rows, sem):
        pltpu.make_async_copy(rows, acc.at[dstb.at[0]], sem).wait()

    iload(0, 0, si0)
    iload(1, 1, si1)
    iload(2, 2, si2)
    iload(3, 3, si3)
    iwait(0, si0)
    gather(0, r0)
    ascatter(0, r0, s0)
    iwait(1, si1)
    gather(1, r1)
    ascatter(1, r1, s1)
    swait(r0, s0)
    iwait(2, si2)
    gather(2, r0)
    ascatter(2, r0, s0)
    iload(4, 0, si0)
    swait(r1, s1)
    iwait(3, si3)
    gather(3, r1)
    ascatter(3, r1, s1)
    iload(5, 1, si1)

    def body(jj, carry):
        j = jj * 4
        swait(r0, s0)
        iwait(0, si0)
        gather(j, r0)
        ascatter(0, r0, s0)
        iload(j + 2, 2, si2)
        swait(r1, s1)
        iwait(1, si1)
        gather(j + 1, r1)
        ascatter(1, r1, s1)
        iload(j + 3, 3, si3)
        swait(r0, s0)
        iwait(2, si2)
        gather(j + 2, r0)
        ascatter(2, r0, s0)

        @pl.when(j + 4 < nchunk)
        def _():
            iload(j + 4, 0, si0)

        swait(r1, s1)
        iwait(3, si3)
        gather(j + 3, r1)
        ascatter(3, r1, s1)

        @pl.when(j + 5 < nchunk)
        def _():
            iload(j + 5, 1, si1)

        return carry

    lax.fori_loop(1, nchunk // 4, body, 0)
    swait(r0, s0)
    swait(r1, s1)
    plsc.subcore_barrier()
    pltpu.sync_copy(acc.at[pl.ds(s * npt, npt)], out_hbm.at[c, pl.ds(s * npt, npt)])


def _message_pass(srcoff, dst_tiled, z2, npad, h):
    mesh = plsc.VectorSubcoreMesh(
        core_axis_name="c", subcore_axis_name="s", num_cores=NC, num_subcores=NS)
    nchunk, ch = dst_tiled.shape[1], dst_tiled.shape[2]
    zflat = z2.reshape(NC * npad, h)
    fn = pl.kernel(
        _scat_body,
        out_type=jax.ShapeDtypeStruct((NC, npad, h), jnp.float32),
        mesh=mesh,
        scratch_types=[
            pltpu.VMEM_SHARED((npad, h), jnp.float32),
            pltpu.VMEM((nchunk, ch), jnp.int32),
            pltpu.VMEM((4, ch), jnp.int32),
            pltpu.VMEM((ch, h), jnp.float32),
            pltpu.VMEM((ch, h), jnp.float32),
        ] + [pltpu.SemaphoreType.DMA] * 7,
    )
    return fn(srcoff, dst_tiled, zflat)


def _ep_body(p_ref, inv_ref, b_ref, o_ref):
    c = pl.program_id(1)
    bb = jnp.where(c == 0, b_ref[0], b_ref[1])
    o_ref[...] = jnp.maximum(p_ref[0] * inv_ref[...] + bb, 0.0)


def _epilogue(pooled, inv, b2, n, u, bn=5000):
    h = u // 2
    grid = (n // bn, NC)
    return pl.pallas_call(
        _ep_body,
        grid=grid,
        in_specs=[
            pl.BlockSpec((1, bn, h), lambda i, c: (c, i, 0)),
            pl.BlockSpec((bn, 1), lambda i, c: (i, 0)),
            pl.BlockSpec((NC, h), lambda i, c: (0, 0)),
        ],
        out_specs=pl.BlockSpec((bn, h), lambda i, c: (i, c)),
        out_shape=jax.ShapeDtypeStruct((n, u), jnp.float32),
    )(pooled, inv, b2)


def kernel(x, edge_index, W, b):
    n, d = x.shape
    u = W.shape[1]
    h = u // 2
    e = edge_index.shape[1]
    src = edge_index[0].astype(jnp.int32)
    dst = edge_index[1].astype(jnp.int32)
    npad = -(-n // (NS * 8)) * (NS * 8)

    nw = NC * NS
    epw = -(-e // (nw * 16)) * 16
    dst_deg = jnp.concatenate(
        [dst, jnp.full((nw * epw - e,), npad - 1, jnp.int32)]).reshape(nw, epw)

    ch = 100
    nch = -(-(e // NS) // (4 * ch)) * 4
    epad = NS * nch * ch - e
    srcp = jnp.concatenate([src, jnp.full((epad,), npad - 1, jnp.int32)])
    dstp = jnp.concatenate([dst, jnp.full((epad,), npad - 1, jnp.int32)])
    srcoff = jnp.stack([srcp, srcp + npad]).reshape(NC, NS, nch, ch)
    dst_sc = dstp.reshape(NS, nch, ch)

    degp = _degree_partials(dst_deg, npad)
    z2, inv = _matmul(degp, x, W, npad)
    pooled = _message_pass(srcoff, dst_sc, z2, npad, h)
    return _epilogue(pooled, inv, b.reshape(NC, h), n, u)


# speedup vs baseline: 2.0206x; 1.0086x over previous
import functools

import jax
import jax.numpy as jnp
from jax import lax
from jax.experimental import pallas as pl
from jax.experimental.pallas import tpu as pltpu
from jax.experimental.pallas import tpu_sc as plsc

NC = 2
NS = 16

def _deg_body(dst_hbm, out_hbm, hist, dstw):
    c = lax.axis_index("c")
    s = lax.axis_index("s")
    w = c * NS + s
    npad = hist.shape[0]
    nch = dstw.shape[0] // 16
    zero16 = jnp.zeros((16,), jnp.float32)

    def zstep(i, carry):
        hist[pl.ds(i * 16, 16)] = zero16
        return carry

    lax.fori_loop(0, npad // 16, zstep, 0)
    pltpu.sync_copy(dst_hbm.at[w], dstw)
    ones = jnp.ones((16,), jnp.float32)

    def step(i, carry):
        idx = dstw[pl.ds(i * 16, 16)]
        plsc.addupdate_scatter(hist, [idx], ones)
        return carry

    lax.fori_loop(0, nch, step, 0)
    pltpu.sync_copy(hist, out_hbm.at[w])


def _degree_partials(dst_tiled, npad):
    mesh = plsc.VectorSubcoreMesh(
        core_axis_name="c", subcore_axis_name="s", num_cores=NC, num_subcores=NS)
    epw = dst_tiled.shape[1]
    fn = pl.kernel(
        _deg_body,
        out_type=jax.ShapeDtypeStruct((NC * NS, npad), jnp.float32),
        mesh=mesh,
        compiler_params=pltpu.CompilerParams(needs_layout_passes=False),
        scratch_types=[
            pltpu.VMEM((npad,), jnp.float32),
            pltpu.VMEM((epw,), jnp.int32),
        ],
    )
    return fn(dst_tiled)


def _mm_body(deg_ref, x_ref, w_ref, z2_ref, inv_ref):
    d = jnp.sum(deg_ref[...], axis=0) + 1.0
    inv = lax.rsqrt(d)
    xn = x_ref[...] * inv[:, None]
    z = jnp.dot(xn, w_ref[...], preferred_element_type=jnp.float32)
    h = z.shape[1] // 2
    z2_ref[0] = z[:, :h]
    z2_ref[1] = z[:, h:]
    inv_ref[...] = inv[:, None]


def _matmul(degp, x, W, npad, bn=2048):
    n, d = x.shape
    u = W.shape[1]
    h = u // 2
    grid = (-(-npad // bn),)
    return pl.pallas_call(
        _mm_body,
        grid=grid,
        in_specs=[
            pl.BlockSpec((NC * NS, bn), lambda i: (0, i)),
            pl.BlockSpec((bn, d), lambda i: (i, 0)),
            pl.BlockSpec((d, u), lambda i: (0, 0)),
        ],
        out_specs=[
            pl.BlockSpec((NC, bn, h), lambda i: (0, i, 0)),
            pl.BlockSpec((bn, 1), lambda i: (i, 0)),
        ],
        out_shape=[
            jax.ShapeDtypeStruct((NC, npad, h), jnp.float32),
            jax.ShapeDtypeStruct((n, 1), jnp.float32),
        ],
    )(degp, x, W)


def _scat_body(srcoff_hbm, dst_hbm, zflat_hbm, out_hbm,
               acc, srcb, dstb, r0, r1, sg, s0, s1, si0, si1, si2, si3):
    c = lax.axis_index("c")
    s = lax.axis_index("s")
    npad = acc.shape[0]
    npt = npad // NS
    nchunk = srcb.shape[0]
    pltpu.sync_copy(zflat_hbm.at[pl.ds(c * npad + s * npt, npt)],
                    acc.at[pl.ds(s * npt, npt)])
    pltpu.sync_copy(srcoff_hbm.at[c, s], srcb)
    plsc.subcore_barrier()

    def iload(j, k, sem):
        pltpu.async_copy(dst_hbm.at[s, j], dstb.at[k], sem)

    def iwait(k, sem):
        pltpu.make_async_copy(dst_hbm.at[s, 0], dstb.at[k], sem).wait()

    def gather(j, rows):
        pltpu.async_copy(zflat_hbm.at[srcb.at[j]], rows, sg).wait()

    def ascatter(k, rows, sem):
        pltpu.async_copy(rows, acc.at[dstb.at[k]], sem, add=True)

    def swait(rows, sem):
        pltpu.make_async_copy(rows, acc.at[dstb.at[0]], sem).wait()

    iload(0, 0, si0)
    iload(1, 1, si1)
    iload(2, 2, si2)
    iload(3, 3, si3)
    iwait(0, si0)
    gather(0, r0)
    ascatter(0, r0, s0)
    iwait(1, si1)
    gather(1, r1)
    ascatter(1, r1, s1)
    swait(r0, s0)
    iwait(2, si2)
    gather(2, r0)
    ascatter(2, r0, s0)
    iload(4, 0, si0)
    swait(r1, s1)
    iwait(3, si3)
    gather(3, r1)
    ascatter(3, r1, s1)
    iload(5, 1, si1)

    def body(jj, carry):
        j = jj * 4
        swait(r0, s0)
        iwait(0, si0)
        gather(j, r0)
        ascatter(0, r0, s0)
        iload(j + 2, 2, si2)
        swait(r1, s1)
        iwait(1, si1)
        gather(j + 1, r1)
        ascatter(1, r1, s1)
        iload(j + 3, 3, si3)
        swait(r0, s0)
        iwait(2, si2)
        gather(j + 2, r0)
        ascatter(2, r0, s0)

        @pl.when(j + 4 < nchunk)
        def _():
            iload(j + 4, 0, si0)

        swait(r1, s1)
        iwait(3, si3)
        gather(j + 3, r1)
        ascatter(3, r1, s1)

        @pl.when(j + 5 < nchunk)
        def _():
            iload(j + 5, 1, si1)

        return carry

    lax.fori_loop(1, nchunk // 4, body, 0)
    swait(r0, s0)
    swait(r1, s1)
    plsc.subcore_barrier()
    pltpu.sync_copy(acc.at[pl.ds(s * npt, npt)], out_hbm.at[c, pl.ds(s * npt, npt)])


def _message_pass(srcoff, dst_tiled, z2, npad, h):
    mesh = plsc.VectorSubcoreMesh(
        core_axis_name="c", subcore_axis_name="s", num_cores=NC, num_subcores=NS)
    nchunk, ch = dst_tiled.shape[1], dst_tiled.shape[2]
    zflat = z2.reshape(NC * npad, h)
    fn = pl.kernel(
        _scat_body,
        out_type=jax.ShapeDtypeStruct((NC, npad, h), jnp.float32),
        mesh=mesh,
        scratch_types=[
            pltpu.VMEM_SHARED((npad, h), jnp.float32),
            pltpu.VMEM((nchunk, ch), jnp.int32),
            pltpu.VMEM((4, ch), jnp.int32),
            pltpu.VMEM((ch, h), jnp.float32),
            pltpu.VMEM((ch, h), jnp.float32),
        ] + [pltpu.SemaphoreType.DMA] * 7,
    )
    return fn(srcoff, dst_tiled, zflat)


def _ep_body(p_ref, inv_ref, b_ref, o_ref):
    c = pl.program_id(1)
    bb = jnp.where(c == 0, b_ref[0], b_ref[1])
    o_ref[...] = jnp.maximum(p_ref[0] * inv_ref[...] + bb, 0.0)


def _epilogue(pooled, inv, b2, n, u, bn=10000):
    h = u // 2
    grid = (n // bn, NC)
    return pl.pallas_call(
        _ep_body,
        grid=grid,
        in_specs=[
            pl.BlockSpec((1, bn, h), lambda i, c: (c, i, 0)),
            pl.BlockSpec((bn, 1), lambda i, c: (i, 0)),
            pl.BlockSpec((NC, h), lambda i, c: (0, 0)),
        ],
        out_specs=pl.BlockSpec((bn, h), lambda i, c: (i, c)),
        out_shape=jax.ShapeDtypeStruct((n, u), jnp.float32),
    )(pooled, inv, b2)


def kernel(x, edge_index, W, b):
    n, d = x.shape
    u = W.shape[1]
    h = u // 2
    e = edge_index.shape[1]
    src = edge_index[0].astype(jnp.int32)
    dst = edge_index[1].astype(jnp.int32)
    npad = -(-n // (NS * 8)) * (NS * 8)

    nw = NC * NS
    epw = -(-e // (nw * 16)) * 16
    dst_deg = jnp.concatenate(
        [dst, jnp.full((nw * epw - e,), npad - 1, jnp.int32)]).reshape(nw, epw)

    ch = 100
    nch = -(-(e // NS) // (4 * ch)) * 4
    epad = NS * nch * ch - e
    srcp = jnp.concatenate([src, jnp.full((epad,), npad - 1, jnp.int32)])
    dstp = jnp.concatenate([dst, jnp.full((epad,), npad - 1, jnp.int32)])
    srcoff = jnp.stack([srcp, srcp + npad]).reshape(NC, NS, nch, ch)
    dst_sc = dstp.reshape(NS, nch, ch)

    degp = _degree_partials(dst_deg, npad)
    z2, inv = _matmul(degp, x, W, npad)
    pooled = _message_pass(srcoff, dst_sc, z2, npad, h)
    return _epilogue(pooled, inv, b.reshape(NC, h), n, u)
